# baseline (speedup 1.0000x reference)
import functools

import jax
import jax.numpy as jnp
from jax import lax
from jax.experimental import pallas as pl
from jax.experimental.pallas import tpu as pltpu
from jax.experimental.pallas import tpu_sc as plsc

_B = 4
_SEQ = 4096
_D = 1024
_HEADS = 16

_NC = 2
_NS = 16
_NW = _NC * _NS
_N_IDS = _B * _SEQ
_PER_W = _N_IDS // _NW
_W_PER_B = _SEQ // _PER_W
_CHUNK = 16
_N_CHUNKS = _PER_W // _CHUNK
_NBUF = 6


def _gather_body(idx_hbm, table_hbm, out_hbm, idx_v, *scratch):
    wid = lax.axis_index("s") * _NC + lax.axis_index("c")
    row_b = wid // _W_PER_B
    col0 = (wid % _W_PER_B) * _PER_W
    pltpu.sync_copy(idx_hbm.at[row_b, pl.ds(col0, _PER_W)], idx_v)

    bufs = scratch[:_NBUF]
    gsems = scratch[_NBUF:2 * _NBUF]
    osems = scratch[2 * _NBUF:]

    def gather(i, b):
        pltpu.make_async_copy(table_hbm.at[idx_v.at[pl.ds(i * _CHUNK, _CHUNK)]],
                              bufs[b], gsems[b]).start()

    def gather_wait(i, b):
        pltpu.make_async_copy(table_hbm.at[idx_v.at[pl.ds(i * _CHUNK, _CHUNK)]],
                              bufs[b], gsems[b]).wait()

    def out_start(i, b):
        pltpu.make_async_copy(bufs[b],
                              out_hbm.at[row_b,
                                         pl.ds(col0 + i * _CHUNK, _CHUNK)],
                              osems[b]).start()

    def out_wait(i, b):
        pltpu.make_async_copy(bufs[b],
                              out_hbm.at[row_b,
                                         pl.ds(col0 + i * _CHUNK, _CHUNK)],
                              osems[b]).wait()

    for b in range(_NBUF):
        gather(b, b)

    def steady(j, _):
        for b in range(_NBUF):
            i = _NBUF * j + b
            gather_wait(i, b)
            out_start(i, b)
            out_wait(i, b)
            gather(i + _NBUF, b)
        return 0

    lax.fori_loop(0, _N_CHUNKS // _NBUF - 1, steady, 0)

    tail = _N_CHUNKS - _NBUF - (_N_CHUNKS % _NBUF)
    for i in range(tail, _N_CHUNKS - _NBUF):
        b = i % _NBUF
        gather_wait(i, b)
        out_start(i, b)
        out_wait(i, b)
        gather(i + _NBUF, b)
    for i in range(_N_CHUNKS - _NBUF, _N_CHUNKS):
        b = i % _NBUF
        gather_wait(i, b)
        out_start(i, b)
    for i in range(_N_CHUNKS - _NBUF, _N_CHUNKS):
        out_wait(i, i % _NBUF)


@jax.jit
def _sc_gather(ids, table):
    mesh = plsc.VectorSubcoreMesh(core_axis_name="c", subcore_axis_name="s")
    f = functools.partial(
        pl.kernel,
        mesh=mesh,
        out_type=jax.ShapeDtypeStruct((_B, _SEQ, _D), jnp.float32),
        scratch_types=(
            [pltpu.VMEM((_PER_W,), jnp.int32)]
            + [pltpu.VMEM((_CHUNK, _D), jnp.float32)] * _NBUF
            + [pltpu.SemaphoreType.DMA] * (2 * _NBUF)
        ),
    )(_gather_body)
    return f(ids, table)


def _mask_body(mask_ref, ext_ref, bias_ref):
    ext_ref[...] = (1.0 - mask_ref[...]) * -10000.0
    bias_ref[...] = jnp.zeros_like(bias_ref)


@jax.jit
def _tc_mask(mask):
    return pl.pallas_call(
        _mask_body,
        out_shape=[
            jax.ShapeDtypeStruct((_B, _SEQ), jnp.float32),
            jax.ShapeDtypeStruct((_B * _HEADS, _SEQ), jnp.float32),
        ],
    )(mask)


def kernel(encoder_input_ids, encoder_attention_mask, embedding_table):
    ids = encoder_input_ids.astype(jnp.int32)
    hidden = _sc_gather(ids, embedding_table)
    ext, bias = _tc_mask(encoder_attention_mask)
    ext = ext.reshape(_B, 1, 1, _SEQ)
    bias = bias.reshape(_B, _HEADS, _SEQ, 1)
    return (hidden, ext, bias)
